# baseline (speedup 1.0000x reference)
import jax
import jax.numpy as jnp
import numpy as np
from jax import lax
from jax.experimental import pallas as pl
from jax.experimental.pallas import tpu as pltpu
from jax.experimental.pallas import tpu_sc as plsc

N = 20000
G = 20
NPAD = 20480
NC = 2
NS = 16
NW = NC * NS
PW = NPAD // NW
NEG_THRESH = 0.3
POS_THRESH = 0.7
EPS = float(np.finfo(np.float32).eps)
LN2 = 0.6931471805599453
SQRT2 = 1.4142135


def _bcast_f(x):
    return jnp.full((16,), x, jnp.float32)


def _bcast_i(x):
    return jnp.full((16,), x, jnp.int32)


def _log16(x):
    bits = lax.bitcast_convert_type(x, jnp.int32)
    e = (bits >> 23) - 127
    m = lax.bitcast_convert_type((bits & 0x7FFFFF) | 0x3F800000, jnp.float32)
    big = m > SQRT2
    m = jnp.where(big, m * 0.5, m)
    e = jnp.where(big, e + 1, e)
    t = (m - 1.0) / (m + 1.0)
    t2 = t * t
    p = jnp.float32(2.0 / 9.0)
    p = p * t2 + jnp.float32(2.0 / 7.0)
    p = p * t2 + jnp.float32(2.0 / 5.0)
    p = p * t2 + jnp.float32(2.0 / 3.0)
    p = p * t2 + jnp.float32(2.0)
    return e.astype(jnp.float32) * jnp.float32(LN2) + t * p


def _wid():
    return lax.axis_index("c") * NS + lax.axis_index("s")


def _p1_body(inp_h, bbox_h, amax_h, aarg_h, cmax_h, cidx_h,
             stg, bbox_v, area_a, accmax, accarg, gmax_v, gidx_v):
    wid = _wid()
    base = wid * PW
    lane = lax.iota(jnp.int32, 16)

    pltpu.sync_copy(inp_h.at[:4, pl.ds(base, PW)], stg)
    pltpu.sync_copy(bbox_h, bbox_v)

    @plsc.parallel_loop(0, PW, 16)
    def _(off):
        sl = pl.ds(off, 16)
        accmax[sl] = _bcast_f(-1.0)
        accarg[sl] = _bcast_i(0)
        area_a[sl] = (stg[2, sl] - stg[0, sl]) * (stg[3, sl] - stg[1, sl])

    def g_body(g, _):
        gi = g.astype(jnp.int32)
        bx0 = plsc.load_gather(bbox_v, [_bcast_i(4 * gi)])
        by0 = plsc.load_gather(bbox_v, [_bcast_i(4 * gi + 1)])
        bx1 = plsc.load_gather(bbox_v, [_bcast_i(4 * gi + 2)])
        by1 = plsc.load_gather(bbox_v, [_bcast_i(4 * gi + 3)])
        area_b = (bx1 - bx0) * (by1 - by0)

        @plsc.parallel_loop(0, PW, 16, unroll=2,
                            carry=(_bcast_f(-2.0), _bcast_i(NPAD)))
        def cand(off, carry):
            gmaxv, gidxv = carry
            sl = pl.ds(off, 16)
            x0 = stg[0, sl]; y0 = stg[1, sl]; x1 = stg[2, sl]; y1 = stg[3, sl]
            tlx = jnp.maximum(x0, bx0)
            tly = jnp.maximum(y0, by0)
            brx = jnp.minimum(x1, bx1)
            bry = jnp.minimum(y1, by1)
            inter = (jnp.maximum(brx - tlx, 0.0) *
                     jnp.maximum(bry - tly, 0.0))
            iou = inter / (area_a[sl] + area_b - inter)
            am = accmax[sl]
            upd = iou > am
            accmax[sl] = jnp.where(upd, iou, am)
            accarg[sl] = jnp.where(upd, _bcast_i(gi), accarg[sl])
            glob = base + off + lane
            upd2 = iou > gmaxv
            return jnp.where(upd2, iou, gmaxv), jnp.where(upd2, glob, gidxv)

        gmaxv, gidxv = cand
        m = jnp.max(gmaxv)
        mi = jnp.min(jnp.where(gmaxv == m, gidxv, _bcast_i(NPAD)))
        onelane = lane == 0
        plsc.store_scatter(gmax_v, [_bcast_i(gi)], _bcast_f(m), mask=onelane)
        plsc.store_scatter(gidx_v, [_bcast_i(gi)], _bcast_i(mi), mask=onelane)
        return 0

    lax.fori_loop(0, G, g_body, 0)

    pltpu.sync_copy(accmax, amax_h.at[pl.ds(base, PW)])
    pltpu.sync_copy(accarg, aarg_h.at[pl.ds(base, PW)])
    pltpu.sync_copy(gmax_v, cmax_h.at[pl.ds(wid * 32, 32)])
    pltpu.sync_copy(gidx_v, cidx_h.at[pl.ds(wid * 32, 32)])


def _p2_body(inp_h, bbox_h, amax_h, aarg_h, cmax_h, cidx_h, out_h,
             stg, bbox_v, accmax, accarg, accfrc,
             allmax_v, allidx_v, parts_v, allparts_v, outv, sparts):
    cid = lax.axis_index("c")
    sid = lax.axis_index("s")
    wid = cid * NS + sid
    base = wid * PW
    lane = lax.iota(jnp.int32, 16)

    pltpu.sync_copy(inp_h.at[:, pl.ds(base, PW)], stg)
    pltpu.sync_copy(bbox_h, bbox_v)
    pltpu.sync_copy(amax_h.at[pl.ds(base, PW)], accmax)
    pltpu.sync_copy(aarg_h.at[pl.ds(base, PW)], accarg)
    pltpu.sync_copy(cmax_h, allmax_v)
    pltpu.sync_copy(cidx_h, allidx_v)

    @plsc.parallel_loop(0, PW, 16)
    def _(off):
        accfrc[pl.ds(off, 16)] = _bcast_i(0)

    onelane = lane == 0
    lane2 = lane + 16
    for g in range(G):
        gcol = _bcast_i(g)
        v0 = plsc.load_gather(allmax_v, [lane * 32 + gcol])
        v1 = plsc.load_gather(allmax_v, [lane2 * 32 + gcol])
        i0 = plsc.load_gather(allidx_v, [lane * 32 + gcol])
        i1 = plsc.load_gather(allidx_v, [lane2 * 32 + gcol])
        m = jnp.maximum(jnp.max(v0), jnp.max(v1))
        mi0 = jnp.min(jnp.where(v0 == m, i0, _bcast_i(NPAD)))
        mi1 = jnp.min(jnp.where(v1 == m, i1, _bcast_i(NPAD)))
        mi = jnp.minimum(mi0, mi1)
        loc = mi - base
        inr = (loc >= 0) & (loc < PW)
        lc = _bcast_i(jnp.clip(loc, 0, PW - 1))
        msk = onelane & inr
        plsc.store_scatter(accarg, [lc], gcol, mask=msk)
        plsc.store_scatter(accfrc, [lc], _bcast_i(1), mask=msk)

    z = _bcast_f(0.0)

    @plsc.parallel_loop(0, PW, 16, carry=(z, z, z, z, z))
    def sums(off, carry):
        sp, sv, slc, sce, sse = carry
        sl = pl.ds(off, 16)
        am = accmax[sl]
        aa = accarg[sl]
        af = accfrc[sl]
        glob = base + off + lane
        real = glob < N
        pos = ((am >= POS_THRESH) | (af == 1)) & real
        neg = (am < NEG_THRESH) & (af == 0) & real
        valid = pos | neg
        posf = jnp.where(pos, 1.0, 0.0).astype(jnp.float32)
        validf = jnp.where(valid, 1.0, 0.0).astype(jnp.float32)

        x0 = stg[0, sl]; y0 = stg[1, sl]; x1 = stg[2, sl]; y1 = stg[3, sl]
        b4 = aa * 4
        sx0 = plsc.load_gather(bbox_v, [b4])
        sy0 = plsc.load_gather(bbox_v, [b4 + 1])
        sx1 = plsc.load_gather(bbox_v, [b4 + 2])
        sy1 = plsc.load_gather(bbox_v, [b4 + 3])

        w = jnp.maximum(x1 - x0, EPS)
        h = jnp.maximum(y1 - y0, EPS)
        cx = x0 + 0.5 * (x1 - x0)
        cy = y0 + 0.5 * (y1 - y0)
        bw = sx1 - sx0
        bh = sy1 - sy0
        bcx = sx0 + 0.5 * bw
        bcy = sy0 + 0.5 * bh
        d0 = (bcx - cx) / w
        d1 = (bcy - cy) / h
        d2 = _log16(bw / w)
        d3 = _log16(bh / h)

        lsum = _bcast_f(0.0)
        for dv, j in ((d0, 4), (d1, 5), (d2, 6), (d3, 7)):
            ad = jnp.abs(dv - stg[j, sl])
            lsum = lsum + jnp.where(ad < 1.0, 0.5 * ad * ad, ad - 0.5)

        a0 = stg[8, sl]
        a1 = stg[9, sl]
        mx = jnp.maximum(a0, a1)
        lz = mx + _log16(1.0 + jnp.exp(jnp.minimum(a0, a1) - mx))
        ce = lz - jnp.where(pos, a1, a0)

        fgd = stg[10, sl] - am
        se = fgd * fgd
        return (sp + posf, sv + validf, slc + lsum * posf,
                sce + ce * validf, sse + se * posf)

    sp, sv, slc, sce, sse = sums

    pv = jnp.where(lane == 0, _bcast_f(jnp.sum(sp)),
         jnp.where(lane == 1, _bcast_f(jnp.sum(sv)),
         jnp.where(lane == 2, _bcast_f(jnp.sum(slc)),
         jnp.where(lane == 3, _bcast_f(jnp.sum(sce)),
         jnp.where(lane == 4, _bcast_f(jnp.sum(sse)), _bcast_f(0.0))))))
    parts_v[...] = pv
    pltpu.sync_copy(parts_v, sparts.at[pl.ds(sid * 16, 16)])
    plsc.subcore_barrier()

    @pl.when(sid == 0)
    def _():
        pltpu.sync_copy(sparts, allparts_v)
        tp = jnp.sum(plsc.load_gather(allparts_v, [lane * 16 + _bcast_i(0)]))
        tv = jnp.sum(plsc.load_gather(allparts_v, [lane * 16 + _bcast_i(1)]))
        tl = jnp.sum(plsc.load_gather(allparts_v, [lane * 16 + _bcast_i(2)]))
        tc = jnp.sum(plsc.load_gather(allparts_v, [lane * 16 + _bcast_i(3)]))
        ts = jnp.sum(plsc.load_gather(allparts_v, [lane * 16 + _bcast_i(4)]))
        outvec = jnp.where(lane == 0, _bcast_f(tp),
                 jnp.where(lane == 1, _bcast_f(tv),
                 jnp.where(lane == 2, _bcast_f(tl),
                 jnp.where(lane == 3, _bcast_f(tc),
                 jnp.where(lane == 4, _bcast_f(ts), _bcast_f(0.0))))))
        outv[...] = outvec
        pltpu.sync_copy(outv, out_h.at[pl.ds(cid * 16, 16)])


_vm = lambda shp, dt: pltpu.VMEM(shp, dt)
_mesh = plsc.VectorSubcoreMesh(core_axis_name="c", subcore_axis_name="s",
                               num_cores=NC)
_f32 = jnp.float32
_i32 = jnp.int32

_p1_call = pl.kernel(
    _p1_body,
    out_type=(jax.ShapeDtypeStruct((NPAD,), _f32),
              jax.ShapeDtypeStruct((NPAD,), _i32),
              jax.ShapeDtypeStruct((NW * 32,), _f32),
              jax.ShapeDtypeStruct((NW * 32,), _i32)),
    mesh=_mesh,
    scratch_types=[
        _vm((4, PW), _f32),
        _vm((96,), _f32),
        _vm((PW,), _f32), _vm((PW,), _f32), _vm((PW,), _i32),
        _vm((32,), _f32), _vm((32,), _i32),
    ],
    compiler_params=pltpu.CompilerParams(needs_layout_passes=False),
)

_p2_call = pl.kernel(
    _p2_body,
    out_type=jax.ShapeDtypeStruct((NC * 16,), _f32),
    mesh=_mesh,
    scratch_types=[
        _vm((11, PW), _f32),
        _vm((96,), _f32),
        _vm((PW,), _f32), _vm((PW,), _i32), _vm((PW,), _i32),
        _vm((NW * 32,), _f32), _vm((NW * 32,), _i32),
        _vm((16,), _f32), _vm((NS * 16,), _f32),
        _vm((16,), _f32),
        pltpu.VMEM_SHARED((NS * 16,), _f32),
    ],
    compiler_params=pltpu.CompilerParams(needs_layout_passes=False),
)


def kernel(anchor, bbox, gt_label, rpn_loc, rpn_score, rpn_fg_score):
    pad = NPAD - N
    anc = jnp.pad(anchor, ((0, pad), (0, 0)))
    rl = jnp.pad(rpn_loc, ((0, pad), (0, 0)))
    rs = jnp.pad(rpn_score, ((0, pad), (0, 0)))
    fg = jnp.pad(rpn_fg_score, (0, pad))
    inp = jnp.concatenate([anc.T, rl.T, rs.T, fg[None, :]], axis=0)
    bboxf = jnp.pad(bbox.reshape(-1), (0, 16))
    amax, aarg, cmax, cidx = _p1_call(inp, bboxf)
    out2 = _p2_call(inp, bboxf, amax, aarg, cmax, cidx)
    tot = out2[:16] + out2[16:]
    npos = jnp.maximum(tot[0], 1.0)
    nval = jnp.maximum(tot[1], 1.0)
    loc_l = tot[2] / npos
    cls_l = tot[3] / nval
    reg_l = tot[4] / npos
    return (loc_l, cls_l, reg_l, loc_l + cls_l + reg_l)


# speedup vs baseline: 1.4699x; 1.4699x over previous
import jax
import jax.numpy as jnp
import numpy as np
from jax import lax
from jax.experimental import pallas as pl
from jax.experimental.pallas import tpu as pltpu
from jax.experimental.pallas import tpu_sc as plsc

N = 20000
G = 20
NPAD = 20480
NW = 16
PW = NPAD // NW
NEG_THRESH = 0.3
POS_THRESH = 0.7
EPS = float(np.finfo(np.float32).eps)
LN2 = 0.6931471805599453
SQRT2 = 1.4142135


def _bcast_f(x):
    return jnp.full((16,), x, jnp.float32)


def _bcast_i(x):
    return jnp.full((16,), x, jnp.int32)


def _log16(x):
    bits = lax.bitcast_convert_type(x, jnp.int32)
    e = (bits >> 23) - 127
    m = lax.bitcast_convert_type((bits & 0x7FFFFF) | 0x3F800000, jnp.float32)
    big = m > SQRT2
    m = jnp.where(big, m * 0.5, m)
    e = jnp.where(big, e + 1, e)
    t = (m - 1.0) / (m + 1.0)
    t2 = t * t
    p = jnp.float32(2.0 / 9.0)
    p = p * t2 + jnp.float32(2.0 / 7.0)
    p = p * t2 + jnp.float32(2.0 / 5.0)
    p = p * t2 + jnp.float32(2.0 / 3.0)
    p = p * t2 + jnp.float32(2.0)
    return e.astype(jnp.float32) * jnp.float32(LN2) + t * p


def _sc_body(inp_h, bbox_h, out_h,
             stg, bbox_v, area_a, accmax, accarg, accfrc,
             gmax_v, gidx_v, allmax_v, allidx_v,
             parts_v, allparts_v, outv,
             smax, sidx, sparts):
    wid = lax.axis_index("s") + lax.axis_index("c") * NW
    base = wid * PW
    lane = lax.iota(jnp.int32, 16)

    pltpu.sync_copy(inp_h.at[:, pl.ds(base, PW)], stg)
    pltpu.sync_copy(bbox_h, bbox_v)

    @plsc.parallel_loop(0, PW, 16)
    def _(off):
        sl = pl.ds(off, 16)
        accmax[sl] = _bcast_f(-1.0)
        accarg[sl] = _bcast_i(0)
        accfrc[sl] = _bcast_i(0)
        area_a[sl] = (stg[2, sl] - stg[0, sl]) * (stg[3, sl] - stg[1, sl])

    def g_body(gp, _):
        ga = (gp * 2).astype(jnp.int32)
        gb = ga + 1
        bxy = [plsc.load_gather(bbox_v, [_bcast_i(4 * g + c)])
               for g in (ga, gb) for c in range(4)]
        ax0, ay0, ax1, ay1, cx0, cy0, cx1, cy1 = bxy
        area_ba = (ax1 - ax0) * (ay1 - ay0)
        area_bb = (cx1 - cx0) * (cy1 - cy0)

        @plsc.parallel_loop(0, PW, 16, unroll=2,
                            carry=(_bcast_f(-2.0), _bcast_i(NPAD),
                                   _bcast_f(-2.0), _bcast_i(NPAD)))
        def cand(off, carry):
            gmaxa, gidxa, gmaxb, gidxb = carry
            sl = pl.ds(off, 16)
            x0 = stg[0, sl]; y0 = stg[1, sl]; x1 = stg[2, sl]; y1 = stg[3, sl]
            aa_v = area_a[sl]
            glob = base + off + lane

            ia = (jnp.maximum(jnp.minimum(x1, ax1) - jnp.maximum(x0, ax0), 0.0) *
                  jnp.maximum(jnp.minimum(y1, ay1) - jnp.maximum(y0, ay0), 0.0))
            ioua = ia / (aa_v + area_ba - ia)
            ib = (jnp.maximum(jnp.minimum(x1, cx1) - jnp.maximum(x0, cx0), 0.0) *
                  jnp.maximum(jnp.minimum(y1, cy1) - jnp.maximum(y0, cy0), 0.0))
            ioub = ib / (aa_v + area_bb - ib)

            am = accmax[sl]
            ag = accarg[sl]
            upda = ioua > am
            am = jnp.where(upda, ioua, am)
            ag = jnp.where(upda, _bcast_i(ga), ag)
            updb = ioub > am
            accmax[sl] = jnp.where(updb, ioub, am)
            accarg[sl] = jnp.where(updb, _bcast_i(gb), ag)

            upd2 = ioua > gmaxa
            upd3 = ioub > gmaxb
            return (jnp.where(upd2, ioua, gmaxa), jnp.where(upd2, glob, gidxa),
                    jnp.where(upd3, ioub, gmaxb), jnp.where(upd3, glob, gidxb))

        gmaxa, gidxa, gmaxb, gidxb = cand
        onelane = lane == 0
        for g, gmaxv, gidxv in ((ga, gmaxa, gidxa), (gb, gmaxb, gidxb)):
            m = jnp.max(gmaxv)
            mi = jnp.min(jnp.where(gmaxv == m, gidxv, _bcast_i(NPAD)))
            plsc.store_scatter(gmax_v, [_bcast_i(g)], _bcast_f(m), mask=onelane)
            plsc.store_scatter(gidx_v, [_bcast_i(g)], _bcast_i(mi), mask=onelane)
        return 0

    lax.fori_loop(0, G // 2, g_body, 0)

    pltpu.sync_copy(gmax_v, smax.at[pl.ds(wid * 32, 32)])
    pltpu.sync_copy(gidx_v, sidx.at[pl.ds(wid * 32, 32)])
    plsc.subcore_barrier()
    pltpu.sync_copy(smax, allmax_v)
    pltpu.sync_copy(sidx, allidx_v)

    onelane = lane == 0
    for g in range(G):
        gcol = _bcast_i(g)
        fidx = lane * 32 + gcol
        vals = plsc.load_gather(allmax_v, [fidx])
        idxs = plsc.load_gather(allidx_v, [fidx])
        m = jnp.max(vals)
        mi = jnp.min(jnp.where(vals == m, idxs, _bcast_i(NPAD)))
        loc = mi - base
        inr = (loc >= 0) & (loc < PW)
        lc = _bcast_i(jnp.clip(loc, 0, PW - 1))
        msk = onelane & inr
        plsc.store_scatter(accarg, [lc], gcol, mask=msk)
        plsc.store_scatter(accfrc, [lc], _bcast_i(1), mask=msk)

    z = _bcast_f(0.0)

    @plsc.parallel_loop(0, PW, 16, carry=(z, z, z, z, z))
    def sums(off, carry):
        sp, sv, slc, sce, sse = carry
        sl = pl.ds(off, 16)
        am = accmax[sl]
        aa = accarg[sl]
        af = accfrc[sl]
        glob = base + off + lane
        real = glob < N
        pos = ((am >= POS_THRESH) | (af == 1)) & real
        neg = (am < NEG_THRESH) & (af == 0) & real
        valid = pos | neg
        posf = jnp.where(pos, 1.0, 0.0).astype(jnp.float32)
        validf = jnp.where(valid, 1.0, 0.0).astype(jnp.float32)

        x0 = stg[0, sl]; y0 = stg[1, sl]; x1 = stg[2, sl]; y1 = stg[3, sl]
        b4 = aa * 4
        sx0 = plsc.load_gather(bbox_v, [b4])
        sy0 = plsc.load_gather(bbox_v, [b4 + 1])
        sx1 = plsc.load_gather(bbox_v, [b4 + 2])
        sy1 = plsc.load_gather(bbox_v, [b4 + 3])

        w = jnp.maximum(x1 - x0, EPS)
        h = jnp.maximum(y1 - y0, EPS)
        cx = x0 + 0.5 * (x1 - x0)
        cy = y0 + 0.5 * (y1 - y0)
        bw = sx1 - sx0
        bh = sy1 - sy0
        bcx = sx0 + 0.5 * bw
        bcy = sy0 + 0.5 * bh
        d0 = (bcx - cx) / w
        d1 = (bcy - cy) / h
        d2 = _log16(bw / w)
        d3 = _log16(bh / h)

        lsum = _bcast_f(0.0)
        for dv, j in ((d0, 4), (d1, 5), (d2, 6), (d3, 7)):
            ad = jnp.abs(dv - stg[j, sl])
            lsum = lsum + jnp.where(ad < 1.0, 0.5 * ad * ad, ad - 0.5)

        a0 = stg[8, sl]
        a1 = stg[9, sl]
        mx = jnp.maximum(a0, a1)
        lz = mx + _log16(1.0 + jnp.exp(jnp.minimum(a0, a1) - mx))
        ce = lz - jnp.where(pos, a1, a0)

        fgd = stg[10, sl] - am
        se = fgd * fgd
        return (sp + posf, sv + validf, slc + lsum * posf,
                sce + ce * validf, sse + se * posf)

    sp, sv, slc, sce, sse = sums

    pv = jnp.where(lane == 0, _bcast_f(jnp.sum(sp)),
         jnp.where(lane == 1, _bcast_f(jnp.sum(sv)),
         jnp.where(lane == 2, _bcast_f(jnp.sum(slc)),
         jnp.where(lane == 3, _bcast_f(jnp.sum(sce)),
         jnp.where(lane == 4, _bcast_f(jnp.sum(sse)), _bcast_f(0.0))))))
    parts_v[...] = pv
    pltpu.sync_copy(parts_v, sparts.at[pl.ds(wid * 16, 16)])
    plsc.subcore_barrier()

    @pl.when(wid == 0)
    def _():
        pltpu.sync_copy(sparts, allparts_v)
        tot_p = jnp.sum(plsc.load_gather(allparts_v, [lane * 16 + _bcast_i(0)]))
        tot_v = jnp.sum(plsc.load_gather(allparts_v, [lane * 16 + _bcast_i(1)]))
        tot_l = jnp.sum(plsc.load_gather(allparts_v, [lane * 16 + _bcast_i(2)]))
        tot_c = jnp.sum(plsc.load_gather(allparts_v, [lane * 16 + _bcast_i(3)]))
        tot_s = jnp.sum(plsc.load_gather(allparts_v, [lane * 16 + _bcast_i(4)]))
        npos = jnp.maximum(_bcast_f(tot_p), 1.0)
        nval = jnp.maximum(_bcast_f(tot_v), 1.0)
        loc_l = _bcast_f(tot_l) / npos
        cls_l = _bcast_f(tot_c) / nval
        reg_l = _bcast_f(tot_s) / npos
        outvec = jnp.where(lane == 0, loc_l,
                 jnp.where(lane == 1, cls_l,
                 jnp.where(lane == 2, reg_l,
                 loc_l + cls_l + reg_l)))
        outv[...] = outvec
        pltpu.sync_copy(outv, out_h)


_vm = lambda shp, dt: pltpu.VMEM(shp, dt)
_sc_call = pl.kernel(
    _sc_body,
    out_type=jax.ShapeDtypeStruct((16,), jnp.float32),
    mesh=plsc.VectorSubcoreMesh(core_axis_name="c", subcore_axis_name="s",
                                num_cores=1),
    scratch_types=[
        _vm((11, PW), jnp.float32),
        _vm((96,), jnp.float32),
        _vm((PW,), jnp.float32),
        _vm((PW,), jnp.float32), _vm((PW,), jnp.int32), _vm((PW,), jnp.int32),
        _vm((32,), jnp.float32), _vm((32,), jnp.int32),
        _vm((NW * 32,), jnp.float32), _vm((NW * 32,), jnp.int32),
        _vm((16,), jnp.float32), _vm((NW * 16,), jnp.float32),
        _vm((16,), jnp.float32),
        pltpu.VMEM_SHARED((NW * 32,), jnp.float32),
        pltpu.VMEM_SHARED((NW * 32,), jnp.int32),
        pltpu.VMEM_SHARED((NW * 16,), jnp.float32),
    ],
    compiler_params=pltpu.CompilerParams(needs_layout_passes=False),
)


def kernel(anchor, bbox, gt_label, rpn_loc, rpn_score, rpn_fg_score):
    allc = jnp.concatenate(
        [anchor, rpn_loc, rpn_score, rpn_fg_score[:, None]], axis=1)
    inp = jnp.pad(allc, ((0, NPAD - N), (0, 0))).T
    bboxf = jnp.pad(bbox.reshape(-1), (0, 16))
    out = _sc_call(inp, bboxf)
    return (out[0], out[1], out[2], out[3])
